# baseline (speedup 1.0000x reference)
import functools

import jax
import jax.numpy as jnp
from jax import lax
from jax.experimental import pallas as pl
from jax.experimental.pallas import tpu as pltpu
from jax.experimental.pallas import tpu_sc as plsc

B, S = 1024, 200
N = B * S
V_ITEM, V_PART = 1000001, 1001
D_ITEM, D_PART, D_SMALL, D_POS, D_MODEL = 64, 16, 16, 32, 256
TOTAL_FEAT = 240

NC, NS = 2, 16
NW = NC * NS
PER_W = N // NW
CHUNK = 400
NCHUNK = PER_W // CHUNK
NPAIR = NCHUNK // 2

_sc_mesh = plsc.VectorSubcoreMesh(core_axis_name="c", subcore_axis_name="s")


def _make_sc_gather(ntok):
    per_w = ntok // NW
    npair = per_w // CHUNK // 2

    @functools.partial(
        pl.kernel,
        mesh=_sc_mesh,
        out_type=(
            jax.ShapeDtypeStruct((ntok, 128), jnp.float32),
            jax.ShapeDtypeStruct((ntok, 128), jnp.float32),
        ),
        scratch_types=[
            pltpu.VMEM((per_w,), jnp.int32),
            pltpu.VMEM((CHUNK, 128), jnp.float32),
            pltpu.VMEM((CHUNK, 128), jnp.float32),
            pltpu.SemaphoreType.DMA,
            pltpu.SemaphoreType.DMA,
            pltpu.SemaphoreType.DMA,
            pltpu.SemaphoreType.DMA,
        ],
    )
    def _sc_gather(item_idx, shift_idx, comb_table,
                   out_item, out_shift,
                   idx_all, rows0, rows1, g0, g1, w0, w1):
        wid = lax.axis_index("s") * NC + lax.axis_index("c")
        base = wid * per_w
        rows = (rows0, rows1)
        gsem = (g0, g1)
        wsem = (w0, w1)

        def pass_over(idx_hbm, table, out_hbm):
            pltpu.sync_copy(idx_hbm.at[pl.ds(base, per_w)], idx_all)

            def start_gather(ci, p):
                idx_sl = idx_all.at[pl.ds(ci * CHUNK, CHUNK)]
                pltpu.async_copy(table.at[idx_sl], rows[p], gsem[p])

            def wait_gather(p):
                pltpu.make_async_copy(
                    table.at[pl.ds(0, CHUNK)], rows[p], gsem[p]).wait()

            def start_wb(ci, p):
                pltpu.async_copy(
                    rows[p], out_hbm.at[pl.ds(base + ci * CHUNK, CHUNK)],
                    wsem[p])

            def wait_wb(p):
                pltpu.make_async_copy(
                    rows[p], out_hbm.at[pl.ds(base, CHUNK)], wsem[p]).wait()

            start_gather(0, 0)

            def body(j, carry):
                wait_gather(0)
                start_gather(2 * j + 1, 1)
                start_wb(2 * j, 0)
                wait_gather(1)
                wait_wb(0)

                @pl.when(j + 1 < npair)
                def _():
                    start_gather(2 * j + 2, 0)
                start_wb(2 * j + 1, 1)
                wait_wb(1)
                return carry

            lax.fori_loop(0, npair, body, 0)

        pass_over(item_idx, comb_table, out_item)
        pass_over(shift_idx, comb_table, out_shift)

    return _sc_gather


NH = N // 2
BH = B // 2
_sc_gather_half = _make_sc_gather(NH)


CHUNK_P = 1600
NCHUNK_P = PER_W // CHUNK_P
NPAIR_P = NCHUNK_P // 2


@functools.partial(
    pl.kernel,
    mesh=_sc_mesh,
    out_type=jax.ShapeDtypeStruct((N, D_PART), jnp.float32),
    scratch_types=[
        pltpu.VMEM((PER_W,), jnp.int32),
        pltpu.VMEM((CHUNK_P, D_PART), jnp.float32),
        pltpu.VMEM((CHUNK_P, D_PART), jnp.float32),
        pltpu.SemaphoreType.DMA,
        pltpu.SemaphoreType.DMA,
        pltpu.SemaphoreType.DMA,
        pltpu.SemaphoreType.DMA,
    ],
    compiler_params=pltpu.CompilerParams(use_tc_tiling_on_sc=False),
)
def _sc_part(part_idx, part_table, out_part,
             idx_all, rows0, rows1, g0, g1, w0, w1):
    wid = lax.axis_index("s") * NC + lax.axis_index("c")
    base = wid * PER_W
    rows = (rows0, rows1)
    gsem = (g0, g1)
    wsem = (w0, w1)

    pltpu.sync_copy(part_idx.at[pl.ds(base, PER_W)], idx_all)

    def start_gather(ci, p):
        idx_sl = idx_all.at[pl.ds(ci * CHUNK_P, CHUNK_P)]
        pltpu.async_copy(part_table.at[idx_sl], rows[p], gsem[p])

    def wait_gather(p):
        pltpu.make_async_copy(
            part_table.at[pl.ds(0, CHUNK_P)], rows[p], gsem[p]).wait()

    def start_wb(ci, p):
        pltpu.async_copy(
            rows[p], out_part.at[pl.ds(base + ci * CHUNK_P, CHUNK_P)], wsem[p])

    def wait_wb(p):
        pltpu.make_async_copy(
            rows[p], out_part.at[pl.ds(base, CHUNK_P)], wsem[p]).wait()

    start_gather(0, 0)

    def body(j, carry):
        wait_gather(0)
        start_gather(2 * j + 1, 1)
        start_wb(2 * j, 0)
        wait_gather(1)
        wait_wb(0)

        @pl.when(j + 1 < NPAIR_P)
        def _():
            start_gather(2 * j + 2, 0)
        start_wb(2 * j + 1, 1)
        wait_wb(1)
        return carry

    lax.fori_loop(0, NPAIR_P, body, 0)


KV = 16384
VPAD = ((V_ITEM + KV - 1) // KV) * KV


def _comb_body(wi_ref, ws_ref, out_ref):
    ii = lax.broadcasted_iota(jnp.int32, (D_ITEM, D_ITEM), 0)
    jj = lax.broadcasted_iota(jnp.int32, (D_ITEM, D_ITEM), 1)
    eye = (ii == jj).astype(jnp.float32)
    cn = (((0,), (0,)), ((), ()))
    ti = lax.dot_general(wi_ref[...], eye, cn,
                         preferred_element_type=jnp.float32)
    ts = lax.dot_general(ws_ref[...], eye, cn,
                         preferred_element_type=jnp.float32)
    out_ref[...] = jnp.concatenate([ti, ts], axis=-1)


def _comb_call(wiT, wsT):
    return pl.pallas_call(
        _comb_body,
        grid=(VPAD // KV,),
        in_specs=[
            pl.BlockSpec((D_ITEM, KV), lambda j: (0, j)),
            pl.BlockSpec((D_ITEM, KV), lambda j: (0, j)),
        ],
        out_specs=pl.BlockSpec((KV, 128), lambda j: (j, 0)),
        out_shape=jax.ShapeDtypeStruct((VPAD, 128), jnp.float32),
        compiler_params=pltpu.CompilerParams(
            dimension_semantics=("parallel",)),
    )(wiT, wsT)


BT = 32
RT = BT * S


def _tc_body(gi_ref, gs_ref, gp_ref, ic_ref, it_ref, el_ref, lg_ref,
             pos_ref, small_ref, wagg_ref, bagg_ref, out_ref):
    gi = gi_ref[...][:, 0:D_ITEM]
    gs = gs_ref[...][:, D_ITEM:128]
    gp = gp_ref[...]
    iota3 = lax.broadcasted_iota(jnp.int32, (1, 1, 3), 2)
    sel_c = (ic_ref[...][:, :, None] == iota3).astype(jnp.float32).reshape(RT, 3)
    sel_t = (it_ref[...][:, :, None] == iota3).astype(jnp.float32).reshape(RT, 3)
    el = el_ref[...].reshape(RT, 1)
    lg = lg_ref[...].reshape(RT, 1)
    small = small_ref[...]
    e_corr = jnp.dot(sel_c, small[0:3], preferred_element_type=jnp.float32)
    e_time = jnp.dot(sel_t, small[3:6], preferred_element_type=jnp.float32)
    e_el = el * small[6][None, :]
    e_lg = lg * small[7][None, :]
    posb = jnp.broadcast_to(pos_ref[...][None], (BT, S, D_POS)).reshape(RT, D_POS)
    feat = jnp.concatenate([gi, gp, e_corr, e_time, e_el, e_lg, gs, posb], axis=-1)
    acc = lax.dot_general(feat, wagg_ref[...], (((1,), (1,)), ((), ())),
                          preferred_element_type=jnp.float32)
    out_ref[...] = (acc + bagg_ref[...]).reshape(BT, S, D_MODEL)


def _tc_body_acc(gi_ref, gs_ref, gp_ref, ic_ref, it_ref, el_ref, lg_ref,
                 pos_ref, small_ref, wagg_ref, bagg_ref, prev_ref, out_ref):
    _tc_body(gi_ref, gs_ref, gp_ref, ic_ref, it_ref, el_ref, lg_ref,
             pos_ref, small_ref, wagg_ref, bagg_ref, out_ref)


def _tc_call(gi_h, gs_h, gp, ic, it, el3, lg3,
             pos, small, W_agg, b_agg2d, off, out_prev=None):
    blk_h = pl.BlockSpec((RT, 128), lambda i: (i, 0))
    blk_p = pl.BlockSpec((RT, D_PART), lambda i: (i + off, 0))
    blk2 = pl.BlockSpec((BT, S), lambda i: (i + off, 0))
    blk31 = pl.BlockSpec((BT, S, 1), lambda i: (i + off, 0, 0))
    full = lambda shape: pl.BlockSpec(shape, lambda i: (0,) * len(shape))
    in_specs = [
        blk_h, blk_h, blk_p,
        blk2, blk2, blk31, blk31,
        full((S, D_POS)), full((8, D_SMALL)),
        full((D_MODEL, TOTAL_FEAT)), full((1, D_MODEL)),
    ]
    operands = [gi_h, gs_h, gp, ic, it, el3, lg3,
                pos, small, W_agg, b_agg2d]
    if out_prev is None:
        body, aliases = _tc_body, {}
    else:
        body, aliases = _tc_body_acc, {11: 0}
        in_specs.append(pl.BlockSpec(memory_space=pl.ANY))
        operands.append(out_prev)
    return pl.pallas_call(
        body,
        grid=(BH // BT,),
        in_specs=in_specs,
        out_specs=pl.BlockSpec((BT, S, D_MODEL), lambda i: (i + off, 0, 0)),
        out_shape=jax.ShapeDtypeStruct((B, S, D_MODEL), jnp.float32),
        input_output_aliases=aliases,
        compiler_params=pltpu.CompilerParams(
            dimension_semantics=("arbitrary",)),
    )(*operands)


def kernel(item_id, part_id, is_correct, timeliness, elapsed_time_norm,
           lag_time_norm, shifted_item_id, text_embedding_batch,
           W_item, W_part, W_correct, W_timeliness, W_elapsed, W_lag,
           W_shifted_item, pos, W_agg, b_agg):
    item_flat = item_id.astype(jnp.int32).reshape(N)
    shift_flat = shifted_item_id.astype(jnp.int32).reshape(N)
    part_flat = part_id.astype(jnp.int32).reshape(N)

    gp16 = _sc_part(part_flat, W_part)
    comb_table = _comb_call(W_item.T, W_shifted_item.T)

    gi0, gs0 = _sc_gather_half(
        item_flat[:NH], shift_flat[:NH], comb_table)
    gi1, gs1 = _sc_gather_half(
        item_flat[NH:], shift_flat[NH:], comb_table)

    small = jnp.concatenate(
        [W_correct, W_timeliness, W_elapsed.T, W_lag.T], axis=0)

    ic = is_correct.astype(jnp.int32)
    it = timeliness.astype(jnp.int32)
    b2 = b_agg.reshape(1, D_MODEL)
    nblk = BH // BT
    out0 = _tc_call(gi0, gs0, gp16, ic, it, elapsed_time_norm,
                    lag_time_norm, pos, small, W_agg, b2, 0)
    out = _tc_call(gi1, gs1, gp16, ic, it, elapsed_time_norm,
                   lag_time_norm, pos, small, W_agg, b2, nblk,
                   out_prev=out0)
    return out


# speedup vs baseline: 1.0027x; 1.0027x over previous
import functools

import jax
import jax.numpy as jnp
from jax import lax
from jax.experimental import pallas as pl
from jax.experimental.pallas import tpu as pltpu
from jax.experimental.pallas import tpu_sc as plsc

B, S = 1024, 200
N = B * S
V_ITEM, V_PART = 1000001, 1001
D_ITEM, D_PART, D_SMALL, D_POS, D_MODEL = 64, 16, 16, 32, 256
TOTAL_FEAT = 240

NC, NS = 2, 16
NW = NC * NS
PER_W = N // NW
CHUNK = 400
NCHUNK = PER_W // CHUNK
NPAIR = NCHUNK // 2

_sc_mesh = plsc.VectorSubcoreMesh(core_axis_name="c", subcore_axis_name="s")


def _make_sc_gather(ntok):
    per_w = ntok // NW
    npair = per_w // CHUNK // 2

    @functools.partial(
        pl.kernel,
        mesh=_sc_mesh,
        out_type=(
            jax.ShapeDtypeStruct((ntok, 128), jnp.float32),
            jax.ShapeDtypeStruct((ntok, 128), jnp.float32),
        ),
        scratch_types=[
            pltpu.VMEM((per_w,), jnp.int32),
            pltpu.VMEM((CHUNK, 128), jnp.float32),
            pltpu.VMEM((CHUNK, 128), jnp.float32),
            pltpu.SemaphoreType.DMA,
            pltpu.SemaphoreType.DMA,
            pltpu.SemaphoreType.DMA,
            pltpu.SemaphoreType.DMA,
        ],
    )
    def _sc_gather(item_idx, shift_idx, comb_table,
                   out_item, out_shift,
                   idx_all, rows0, rows1, g0, g1, w0, w1):
        wid = lax.axis_index("s") * NC + lax.axis_index("c")
        base = wid * per_w
        rows = (rows0, rows1)
        gsem = (g0, g1)
        wsem = (w0, w1)

        def pass_over(idx_hbm, table, out_hbm):
            pltpu.sync_copy(idx_hbm.at[pl.ds(base, per_w)], idx_all)

            def start_gather(ci, p):
                idx_sl = idx_all.at[pl.ds(ci * CHUNK, CHUNK)]
                pltpu.async_copy(table.at[idx_sl], rows[p], gsem[p])

            def wait_gather(p):
                pltpu.make_async_copy(
                    table.at[pl.ds(0, CHUNK)], rows[p], gsem[p]).wait()

            def start_wb(ci, p):
                pltpu.async_copy(
                    rows[p], out_hbm.at[pl.ds(base + ci * CHUNK, CHUNK)],
                    wsem[p])

            def wait_wb(p):
                pltpu.make_async_copy(
                    rows[p], out_hbm.at[pl.ds(base, CHUNK)], wsem[p]).wait()

            start_gather(0, 0)

            def body(j, carry):
                wait_gather(0)
                start_gather(2 * j + 1, 1)
                start_wb(2 * j, 0)
                wait_gather(1)
                wait_wb(0)

                @pl.when(j + 1 < npair)
                def _():
                    start_gather(2 * j + 2, 0)
                start_wb(2 * j + 1, 1)
                wait_wb(1)
                return carry

            lax.fori_loop(0, npair, body, 0)

        pass_over(item_idx, comb_table, out_item)
        pass_over(shift_idx, comb_table, out_shift)

    return _sc_gather


_sc_gather_full = _make_sc_gather(N)


CHUNK_P = 1600
NCHUNK_P = PER_W // CHUNK_P
NPAIR_P = NCHUNK_P // 2


@functools.partial(
    pl.kernel,
    mesh=_sc_mesh,
    out_type=jax.ShapeDtypeStruct((N, D_PART), jnp.float32),
    scratch_types=[
        pltpu.VMEM((PER_W,), jnp.int32),
        pltpu.VMEM((CHUNK_P, D_PART), jnp.float32),
        pltpu.VMEM((CHUNK_P, D_PART), jnp.float32),
        pltpu.SemaphoreType.DMA,
        pltpu.SemaphoreType.DMA,
        pltpu.SemaphoreType.DMA,
        pltpu.SemaphoreType.DMA,
    ],
    compiler_params=pltpu.CompilerParams(use_tc_tiling_on_sc=False),
)
def _sc_part(part_idx, part_table, out_part,
             idx_all, rows0, rows1, g0, g1, w0, w1):
    wid = lax.axis_index("s") * NC + lax.axis_index("c")
    base = wid * PER_W
    rows = (rows0, rows1)
    gsem = (g0, g1)
    wsem = (w0, w1)

    pltpu.sync_copy(part_idx.at[pl.ds(base, PER_W)], idx_all)

    def start_gather(ci, p):
        idx_sl = idx_all.at[pl.ds(ci * CHUNK_P, CHUNK_P)]
        pltpu.async_copy(part_table.at[idx_sl], rows[p], gsem[p])

    def wait_gather(p):
        pltpu.make_async_copy(
            part_table.at[pl.ds(0, CHUNK_P)], rows[p], gsem[p]).wait()

    def start_wb(ci, p):
        pltpu.async_copy(
            rows[p], out_part.at[pl.ds(base + ci * CHUNK_P, CHUNK_P)], wsem[p])

    def wait_wb(p):
        pltpu.make_async_copy(
            rows[p], out_part.at[pl.ds(base, CHUNK_P)], wsem[p]).wait()

    start_gather(0, 0)

    def body(j, carry):
        wait_gather(0)
        start_gather(2 * j + 1, 1)
        start_wb(2 * j, 0)
        wait_gather(1)
        wait_wb(0)

        @pl.when(j + 1 < NPAIR_P)
        def _():
            start_gather(2 * j + 2, 0)
        start_wb(2 * j + 1, 1)
        wait_wb(1)
        return carry

    lax.fori_loop(0, NPAIR_P, body, 0)


KV = 16384
VPAD = ((V_ITEM + KV - 1) // KV) * KV


def _comb_body(wi_ref, ws_ref, out_ref):
    ii = lax.broadcasted_iota(jnp.int32, (D_ITEM, D_ITEM), 0)
    jj = lax.broadcasted_iota(jnp.int32, (D_ITEM, D_ITEM), 1)
    eye = (ii == jj).astype(jnp.float32)
    cn = (((0,), (0,)), ((), ()))
    ti = lax.dot_general(wi_ref[...], eye, cn,
                         preferred_element_type=jnp.float32)
    ts = lax.dot_general(ws_ref[...], eye, cn,
                         preferred_element_type=jnp.float32)
    out_ref[...] = jnp.concatenate([ti, ts], axis=-1)


def _comb_call(wiT, wsT):
    return pl.pallas_call(
        _comb_body,
        grid=(VPAD // KV,),
        in_specs=[
            pl.BlockSpec((D_ITEM, KV), lambda j: (0, j)),
            pl.BlockSpec((D_ITEM, KV), lambda j: (0, j)),
        ],
        out_specs=pl.BlockSpec((KV, 128), lambda j: (j, 0)),
        out_shape=jax.ShapeDtypeStruct((VPAD, 128), jnp.float32),
        compiler_params=pltpu.CompilerParams(
            dimension_semantics=("parallel",)),
    )(wiT, wsT)


BT = 32
RT = BT * S


def _tc_body(gi_ref, gs_ref, gp_ref, ic_ref, it_ref, el_ref, lg_ref,
             pos_ref, small_ref, wagg_ref, bagg_ref, out_ref):
    gi = gi_ref[...][:, 0:D_ITEM]
    gs = gs_ref[...][:, D_ITEM:128]
    gp = gp_ref[...]
    iota3 = lax.broadcasted_iota(jnp.int32, (1, 1, 3), 2)
    sel_c = (ic_ref[...][:, :, None] == iota3).astype(jnp.float32).reshape(RT, 3)
    sel_t = (it_ref[...][:, :, None] == iota3).astype(jnp.float32).reshape(RT, 3)
    el = el_ref[...].reshape(RT, 1)
    lg = lg_ref[...].reshape(RT, 1)
    small = small_ref[...]
    e_corr = jnp.dot(sel_c, small[0:3], preferred_element_type=jnp.float32)
    e_time = jnp.dot(sel_t, small[3:6], preferred_element_type=jnp.float32)
    e_el = el * small[6][None, :]
    e_lg = lg * small[7][None, :]
    posb = jnp.broadcast_to(pos_ref[...][None], (BT, S, D_POS)).reshape(RT, D_POS)
    feat = jnp.concatenate([gi, gp, e_corr, e_time, e_el, e_lg, gs, posb], axis=-1)
    acc = lax.dot_general(feat, wagg_ref[...], (((1,), (1,)), ((), ())),
                          preferred_element_type=jnp.float32)
    out_ref[...] = (acc + bagg_ref[...]).reshape(BT, S, D_MODEL)


def _tc_body_acc(gi_ref, gs_ref, gp_ref, ic_ref, it_ref, el_ref, lg_ref,
                 pos_ref, small_ref, wagg_ref, bagg_ref, prev_ref, out_ref):
    _tc_body(gi_ref, gs_ref, gp_ref, ic_ref, it_ref, el_ref, lg_ref,
             pos_ref, small_ref, wagg_ref, bagg_ref, out_ref)


def _tc_call(gi_h, gs_h, gp, ic, it, el3, lg3,
             pos, small, W_agg, b_agg2d, off, out_prev=None):
    blk_h = pl.BlockSpec((RT, 128), lambda i: (i, 0))
    blk_p = pl.BlockSpec((RT, D_PART), lambda i: (i + off, 0))
    blk2 = pl.BlockSpec((BT, S), lambda i: (i + off, 0))
    blk31 = pl.BlockSpec((BT, S, 1), lambda i: (i + off, 0, 0))
    full = lambda shape: pl.BlockSpec(shape, lambda i: (0,) * len(shape))
    in_specs = [
        blk_h, blk_h, blk_p,
        blk2, blk2, blk31, blk31,
        full((S, D_POS)), full((8, D_SMALL)),
        full((D_MODEL, TOTAL_FEAT)), full((1, D_MODEL)),
    ]
    operands = [gi_h, gs_h, gp, ic, it, el3, lg3,
                pos, small, W_agg, b_agg2d]
    if out_prev is None:
        body, aliases = _tc_body, {}
    else:
        body, aliases = _tc_body_acc, {11: 0}
        in_specs.append(pl.BlockSpec(memory_space=pl.ANY))
        operands.append(out_prev)
    return pl.pallas_call(
        body,
        grid=(gi_h.shape[0] // RT,),
        in_specs=in_specs,
        out_specs=pl.BlockSpec((BT, S, D_MODEL), lambda i: (i + off, 0, 0)),
        out_shape=jax.ShapeDtypeStruct((B, S, D_MODEL), jnp.float32),
        input_output_aliases=aliases,
        compiler_params=pltpu.CompilerParams(
            dimension_semantics=("arbitrary",)),
    )(*operands)


def kernel(item_id, part_id, is_correct, timeliness, elapsed_time_norm,
           lag_time_norm, shifted_item_id, text_embedding_batch,
           W_item, W_part, W_correct, W_timeliness, W_elapsed, W_lag,
           W_shifted_item, pos, W_agg, b_agg):
    item_flat = item_id.astype(jnp.int32).reshape(N)
    shift_flat = shifted_item_id.astype(jnp.int32).reshape(N)
    part_flat = part_id.astype(jnp.int32).reshape(N)

    gp16 = _sc_part(part_flat, W_part)
    comb_table = _comb_call(W_item.T, W_shifted_item.T)

    gi128, gs128 = _sc_gather_full(item_flat, shift_flat, comb_table)

    small = jnp.concatenate(
        [W_correct, W_timeliness, W_elapsed.T, W_lag.T], axis=0)

    out = _tc_call(
        gi128, gs128, gp16,
        is_correct.astype(jnp.int32), timeliness.astype(jnp.int32),
        elapsed_time_norm, lag_time_norm,
        pos, small, W_agg, b_agg.reshape(1, D_MODEL), 0)
    return out


# speedup vs baseline: 1.0029x; 1.0001x over previous
import functools

import jax
import jax.numpy as jnp
from jax import lax
from jax.experimental import pallas as pl
from jax.experimental.pallas import tpu as pltpu
from jax.experimental.pallas import tpu_sc as plsc

B, S = 1024, 200
N = B * S
V_ITEM, V_PART = 1000001, 1001
D_ITEM, D_PART, D_SMALL, D_POS, D_MODEL = 64, 16, 16, 32, 256
TOTAL_FEAT = 240

NC, NS = 2, 16
NW = NC * NS
PER_W = N // NW
CHUNK = 400
NCHUNK = PER_W // CHUNK
NPAIR = NCHUNK // 2

_sc_mesh = plsc.VectorSubcoreMesh(core_axis_name="c", subcore_axis_name="s")


def _make_sc_gather(ntok):
    per_w = ntok // NW
    npair = per_w // CHUNK // 2

    @functools.partial(
        pl.kernel,
        mesh=_sc_mesh,
        out_type=(
            jax.ShapeDtypeStruct((ntok, 128), jnp.float32),
            jax.ShapeDtypeStruct((ntok, 128), jnp.float32),
        ),
        scratch_types=[
            pltpu.VMEM((per_w,), jnp.int32),
            pltpu.VMEM((CHUNK, 128), jnp.float32),
            pltpu.VMEM((CHUNK, 128), jnp.float32),
            pltpu.SemaphoreType.DMA,
            pltpu.SemaphoreType.DMA,
            pltpu.SemaphoreType.DMA,
            pltpu.SemaphoreType.DMA,
        ],
    )
    def _sc_gather(item_idx, shift_idx, comb_table,
                   out_item, out_shift,
                   idx_all, rows0, rows1, g0, g1, w0, w1):
        wid = lax.axis_index("s") * NC + lax.axis_index("c")
        base = wid * per_w
        rows = (rows0, rows1)
        gsem = (g0, g1)
        wsem = (w0, w1)

        def pass_over(idx_hbm, table, out_hbm):
            pltpu.sync_copy(idx_hbm.at[pl.ds(base, per_w)], idx_all)

            def start_gather(ci, p):
                idx_sl = idx_all.at[pl.ds(ci * CHUNK, CHUNK)]
                pltpu.async_copy(table.at[idx_sl], rows[p], gsem[p])

            def wait_gather(p):
                pltpu.make_async_copy(
                    table.at[pl.ds(0, CHUNK)], rows[p], gsem[p]).wait()

            def start_wb(ci, p):
                pltpu.async_copy(
                    rows[p], out_hbm.at[pl.ds(base + ci * CHUNK, CHUNK)],
                    wsem[p])

            def wait_wb(p):
                pltpu.make_async_copy(
                    rows[p], out_hbm.at[pl.ds(base, CHUNK)], wsem[p]).wait()

            start_gather(0, 0)

            def body(j, carry):
                wait_gather(0)
                start_gather(2 * j + 1, 1)
                start_wb(2 * j, 0)
                wait_gather(1)
                wait_wb(0)

                @pl.when(j + 1 < npair)
                def _():
                    start_gather(2 * j + 2, 0)
                start_wb(2 * j + 1, 1)
                wait_wb(1)
                return carry

            lax.fori_loop(0, npair, body, 0)

        pass_over(item_idx, comb_table, out_item)
        pass_over(shift_idx, comb_table, out_shift)

    return _sc_gather


_sc_gather_full = _make_sc_gather(N)


CHUNK_P = 1600
NCHUNK_P = PER_W // CHUNK_P
NPAIR_P = NCHUNK_P // 2


@functools.partial(
    pl.kernel,
    mesh=_sc_mesh,
    out_type=jax.ShapeDtypeStruct((N, D_PART), jnp.float32),
    scratch_types=[
        pltpu.VMEM((PER_W,), jnp.int32),
        pltpu.VMEM((CHUNK_P, D_PART), jnp.float32),
        pltpu.VMEM((CHUNK_P, D_PART), jnp.float32),
        pltpu.SemaphoreType.DMA,
        pltpu.SemaphoreType.DMA,
        pltpu.SemaphoreType.DMA,
        pltpu.SemaphoreType.DMA,
    ],
    compiler_params=pltpu.CompilerParams(use_tc_tiling_on_sc=False),
)
def _sc_part(part_idx, part_table, out_part,
             idx_all, rows0, rows1, g0, g1, w0, w1):
    wid = lax.axis_index("s") * NC + lax.axis_index("c")
    base = wid * PER_W
    rows = (rows0, rows1)
    gsem = (g0, g1)
    wsem = (w0, w1)

    pltpu.sync_copy(part_idx.at[pl.ds(base, PER_W)], idx_all)

    def start_gather(ci, p):
        idx_sl = idx_all.at[pl.ds(ci * CHUNK_P, CHUNK_P)]
        pltpu.async_copy(part_table.at[idx_sl], rows[p], gsem[p])

    def wait_gather(p):
        pltpu.make_async_copy(
            part_table.at[pl.ds(0, CHUNK_P)], rows[p], gsem[p]).wait()

    def start_wb(ci, p):
        pltpu.async_copy(
            rows[p], out_part.at[pl.ds(base + ci * CHUNK_P, CHUNK_P)], wsem[p])

    def wait_wb(p):
        pltpu.make_async_copy(
            rows[p], out_part.at[pl.ds(base, CHUNK_P)], wsem[p]).wait()

    start_gather(0, 0)

    def body(j, carry):
        wait_gather(0)
        start_gather(2 * j + 1, 1)
        start_wb(2 * j, 0)
        wait_gather(1)
        wait_wb(0)

        @pl.when(j + 1 < NPAIR_P)
        def _():
            start_gather(2 * j + 2, 0)
        start_wb(2 * j + 1, 1)
        wait_wb(1)
        return carry

    lax.fori_loop(0, NPAIR_P, body, 0)


KV = 16384
VPAD = ((V_ITEM + KV - 1) // KV) * KV


def _comb_body(wi_ref, ws_ref, out_ref):
    ii = lax.broadcasted_iota(jnp.int32, (D_ITEM, D_ITEM), 0)
    jj = lax.broadcasted_iota(jnp.int32, (D_ITEM, D_ITEM), 1)
    eye = (ii == jj).astype(jnp.float32)
    cn = (((0,), (0,)), ((), ()))
    ti = lax.dot_general(wi_ref[...], eye, cn,
                         preferred_element_type=jnp.float32)
    ts = lax.dot_general(ws_ref[...], eye, cn,
                         preferred_element_type=jnp.float32)
    out_ref[...] = jnp.concatenate([ti, ts], axis=-1)


def _comb_call(wiT, wsT):
    return pl.pallas_call(
        _comb_body,
        grid=(VPAD // KV,),
        in_specs=[
            pl.BlockSpec((D_ITEM, KV), lambda j: (0, j)),
            pl.BlockSpec((D_ITEM, KV), lambda j: (0, j)),
        ],
        out_specs=pl.BlockSpec((KV, 128), lambda j: (j, 0)),
        out_shape=jax.ShapeDtypeStruct((VPAD, 128), jnp.float32),
        compiler_params=pltpu.CompilerParams(
            dimension_semantics=("parallel",)),
    )(wiT, wsT)


BT = 32
RT = BT * S


def _tc_body(gi_ref, gs_ref, gp_ref, ic_ref, it_ref, el_ref, lg_ref,
             pos_ref, small_ref, wagg_ref, bagg_ref, out_ref):
    gi = gi_ref[...][:, 0:D_ITEM]
    gs = gs_ref[...][:, D_ITEM:128]
    gp = gp_ref[...]
    iota3 = lax.broadcasted_iota(jnp.int32, (1, 1, 3), 2)
    sel_c = (ic_ref[...][:, :, None] == iota3).astype(jnp.float32).reshape(RT, 3)
    sel_t = (it_ref[...][:, :, None] == iota3).astype(jnp.float32).reshape(RT, 3)
    el = el_ref[...].reshape(RT, 1)
    lg = lg_ref[...].reshape(RT, 1)
    small = small_ref[...]
    e_corr = jnp.dot(sel_c, small[0:3], preferred_element_type=jnp.float32)
    e_time = jnp.dot(sel_t, small[3:6], preferred_element_type=jnp.float32)
    e_el = el * small[6][None, :]
    e_lg = lg * small[7][None, :]
    posb = jnp.broadcast_to(pos_ref[...][None], (BT, S, D_POS)).reshape(RT, D_POS)
    feat = jnp.concatenate([gi, gp, e_corr, e_time, e_el, e_lg, gs, posb], axis=-1)
    acc = lax.dot_general(feat, wagg_ref[...], (((1,), (1,)), ((), ())),
                          preferred_element_type=jnp.float32)
    out_ref[...] = (acc + bagg_ref[...]).reshape(BT, S, D_MODEL)


def _tc_call(gi128, gs128, gp16, ic, it, el3, lg3,
             pos, small, W_agg, b_agg2d):
    blk_h = pl.BlockSpec((RT, 128), lambda i: (i, 0))
    blk_p = pl.BlockSpec((RT, D_PART), lambda i: (i, 0))
    blk2 = pl.BlockSpec((BT, S), lambda i: (i, 0))
    blk31 = pl.BlockSpec((BT, S, 1), lambda i: (i, 0, 0))
    full = lambda shape: pl.BlockSpec(shape, lambda i: (0,) * len(shape))
    return pl.pallas_call(
        _tc_body,
        grid=(B // BT,),
        in_specs=[
            blk_h, blk_h, blk_p,
            blk2, blk2, blk31, blk31,
            full((S, D_POS)), full((8, D_SMALL)),
            full((D_MODEL, TOTAL_FEAT)), full((1, D_MODEL)),
        ],
        out_specs=pl.BlockSpec((BT, S, D_MODEL), lambda i: (i, 0, 0)),
        out_shape=jax.ShapeDtypeStruct((B, S, D_MODEL), jnp.float32),
        compiler_params=pltpu.CompilerParams(
            dimension_semantics=("arbitrary",)),
    )(gi128, gs128, gp16, ic, it, el3, lg3,
      pos, small, W_agg, b_agg2d)


def kernel(item_id, part_id, is_correct, timeliness, elapsed_time_norm,
           lag_time_norm, shifted_item_id, text_embedding_batch,
           W_item, W_part, W_correct, W_timeliness, W_elapsed, W_lag,
           W_shifted_item, pos, W_agg, b_agg):
    item_flat = item_id.astype(jnp.int32).reshape(N)
    shift_flat = shifted_item_id.astype(jnp.int32).reshape(N)
    part_flat = part_id.astype(jnp.int32).reshape(N)

    gp16 = _sc_part(part_flat, W_part)
    comb_table = _comb_call(W_item.T, W_shifted_item.T)

    gi128, gs128 = _sc_gather_full(item_flat, shift_flat, comb_table)

    small = jnp.concatenate(
        [W_correct, W_timeliness, W_elapsed.T, W_lag.T], axis=0)

    out = _tc_call(
        gi128, gs128, gp16,
        is_correct.astype(jnp.int32), timeliness.astype(jnp.int32),
        elapsed_time_norm, lag_time_norm,
        pos, small, W_agg, b_agg.reshape(1, D_MODEL))
    return out
